# baseline (speedup 1.0000x reference)
import jax
import jax.numpy as jnp
from jax.experimental import pallas as pl


def _dg(a, b, ca, cb):
    return jax.lax.dot_general(a, b, (((ca,), (cb,)), ((), ())),
                               preferred_element_type=jnp.float32)


def _fwd(state, wr1, wrb1, wr2, wrb2, wh1, whb1, wh2, whb2,
         c1_rel, c1b, c1_root, c2_rel, c2b, c2_root,
         vw1, vb1, vw2, vb2, vw3, vb3, out):
    relu = jax.nn.relu
    x = state[:]

    self_s = x[:, 0, :6]
    hum = x[:, :, 6:]

    r = relu(_dg(self_s, wr1[:], 1, 1) + wrb1[:])
    r = relu(_dg(r, wr2[:], 1, 1) + wrb2[:])

    h = relu(_dg(hum, wh1[:], 2, 1) + whb1[:])
    h = relu(_dg(h, wh2[:], 2, 1) + whb2[:])

    s1 = h.sum(axis=1) + r

    comb1 = c1_root[:] - c1_rel[:]
    t1 = _dg(s1, c1_rel[:], 1, 1) + c1b[:]
    x1r = relu(_dg(r, comb1, 1, 1) + t1)
    x1h = relu(_dg(h, comb1, 2, 1) + t1[:, None, :])

    s2 = x1h.sum(axis=1) + x1r

    comb2 = c2_root[:] - c2_rel[:]
    x2 = relu(_dg(x1r, comb2, 1, 1) + _dg(s2, c2_rel[:], 1, 1) + c2b[:])

    v = relu(_dg(x2, vw1[:], 1, 1) + vb1[:])
    v = relu(_dg(v, vw2[:], 1, 1) + vb2[:])
    out[:] = (v * vw3[:]).sum(axis=1, keepdims=True) + vb3[:]


def kernel(state, dropout, wr_w1, wr_b1, wr_w2, wr_b2, wh_w1, wh_b1, wh_w2,
           wh_b2, c1_rel_w, c1_rel_b, c1_root_w, c2_rel_w, c2_rel_b,
           c2_root_w, v_w1, v_b1, v_w2, v_b2, v_w3, v_b3):
    B = state.shape[0]
    row = lambda b: b.reshape(1, -1)
    args = (state,
            wr_w1, row(wr_b1), wr_w2, row(wr_b2),
            wh_w1, row(wh_b1), wh_w2, row(wh_b2),
            c1_rel_w, row(c1_rel_b), c1_root_w,
            c2_rel_w, row(c2_rel_b), c2_root_w,
            v_w1, row(v_b1), v_w2, row(v_b2), v_w3, row(v_b3))
    return pl.pallas_call(
        _fwd,
        out_shape=jax.ShapeDtypeStruct((B, 1), jnp.float32),
    )(*args)


# speedup vs baseline: 1.3807x; 1.3807x over previous
import jax
import jax.numpy as jnp
from jax.experimental import pallas as pl

_B = 64
_N = 128
_P = _N // 2

_SECTIONS = (
    ("wr1", (6, 64)), ("wrb1", (1, 64)), ("wr2", (64, 32)), ("wrb2", (1, 32)),
    ("w1p", (14, 128)), ("wb1p", (1, 128)),
    ("w2p", (128, 64)), ("wb2p", (1, 64)),
    ("c1p", (64, 104)), ("comb1", (32, 52)), ("rel1", (32, 52)),
    ("c1b", (1, 52)),
    ("comb2", (52, 32)), ("rel2", (52, 32)), ("c2b", (1, 32)),
    ("vw1", (32, 128)), ("vb1", (1, 128)), ("vw2", (128, 64)),
    ("vb2", (1, 64)), ("vw3", (64, 1)), ("vb3", (1, 1)),
)


def _offsets():
    offs, o = {}, 0
    for name, (rows, cols) in _SECTIONS:
        offs[name] = o
        o += rows + ((-rows) % 8)
    return offs, o


_OFFS, _BLOB_ROWS = _offsets()


def _fwd(self_s, hum, wb, out):
    f32 = jnp.float32
    bf16 = jnp.bfloat16
    relu = jax.nn.relu

    sec = {name: wb[_OFFS[name]:_OFFS[name] + r, :c]
           for name, (r, c) in _SECTIONS}

    def dot(a, b):
        ah = a.astype(bf16)
        al = (a - ah.astype(f32)).astype(bf16)
        bh = b.astype(bf16)
        bl = (b - bh.astype(f32)).astype(bf16)
        d = lambda x, y: jax.lax.dot_general(
            x, y, (((1,), (0,)), ((), ())), preferred_element_type=f32)
        return d(ah, bh) + d(al, bh) + d(ah, bl)

    r = relu(dot(relu(dot(self_s[:], sec["wr1"]) + sec["wrb1"]),
                 sec["wr2"]) + sec["wrb2"])

    h1 = relu(dot(hum[:], sec["w1p"]) + sec["wb1p"])
    hf = relu(dot(h1, sec["w2p"]) + sec["wb2p"])

    hf3 = hf.reshape(_B, _P, 64)
    sall = hf3.sum(axis=1)
    s1 = sall[:, :32] + sall[:, 32:] - hf3[:, _P - 1, 32:] + r

    t1 = dot(s1, sec["rel1"]) + sec["c1b"]
    x1r = relu(dot(r, sec["comb1"]) + t1)
    t1p = jnp.concatenate([t1, t1], axis=1)
    x1 = relu((dot(hf, sec["c1p"])).reshape(_B, _P, 104)
              + t1p[:, None, :])

    zall = x1.sum(axis=1)
    s2 = zall[:, :52] + zall[:, 52:] - x1[:, _P - 1, 52:] + x1r

    x2 = relu(dot(x1r, sec["comb2"]) + dot(s2, sec["rel2"]) + sec["c2b"])

    v = relu(dot(x2, sec["vw1"]) + sec["vb1"])
    v = relu(dot(v, sec["vw2"]) + sec["vb2"])
    out[:] = dot(v, sec["vw3"]) + sec["vb3"]


def _blockdiag2(w):
    r, c = w.shape
    z = jnp.zeros((2 * r, 2 * c), w.dtype)
    return z.at[:r, :c].set(w).at[r:, c:].set(w)


def kernel(state, dropout, wr_w1, wr_b1, wr_w2, wr_b2, wh_w1, wh_b1, wh_w2,
           wh_b2, c1_rel_w, c1_rel_b, c1_root_w, c2_rel_w, c2_rel_b,
           c2_root_w, v_w1, v_b1, v_w2, v_b2, v_w3, v_b3):
    f32 = jnp.float32
    B, A, _ = state.shape

    self_s = state[:, 0, :6]
    hum = state[:, :, 6:]
    hum = jnp.pad(hum, ((0, 0), (0, _N - A), (0, 0)))
    hum = hum.reshape(B * _P, 14)

    row = lambda b: b.reshape(1, -1)
    comb1 = (c1_root_w - c1_rel_w).T
    vals = {
        "wr1": wr_w1.T, "wrb1": row(wr_b1),
        "wr2": wr_w2.T, "wrb2": row(wr_b2),
        "w1p": _blockdiag2(wh_w1.T), "wb1p": jnp.tile(row(wh_b1), (1, 2)),
        "w2p": _blockdiag2(wh_w2.T), "wb2p": jnp.tile(row(wh_b2), (1, 2)),
        "c1p": _blockdiag2(comb1), "comb1": comb1, "rel1": c1_rel_w.T,
        "c1b": row(c1_rel_b),
        "comb2": (c2_root_w - c2_rel_w).T, "rel2": c2_rel_w.T,
        "c2b": row(c2_rel_b),
        "vw1": v_w1.T, "vb1": row(v_b1), "vw2": v_w2.T, "vb2": row(v_b2),
        "vw3": v_w3.T, "vb3": row(v_b3),
    }
    pad8 = lambda a: jnp.pad(
        a, ((0, (-a.shape[0]) % 8), (0, 128 - a.shape[1])))
    wb = jnp.concatenate([pad8(vals[n]) for n, _ in _SECTIONS], axis=0)

    return pl.pallas_call(
        _fwd,
        out_shape=jax.ShapeDtypeStruct((B, 1), f32),
    )(self_s, hum, wb)
